# baseline (speedup 1.0000x reference)
import jax
import jax.numpy as jnp
from jax.experimental import pallas as pl
from jax.experimental.pallas import tpu as pltpu


def _smooth_l1(d):
    return jnp.where(d <= 1.0 / 9.0, 0.5 * 9.0 * d * d, d - 0.5 / 9.0)


def _body(clas_hbm, regs_ref, anc_ref, tcol_ref, trow_ref, out_ref,
          buf_ref, sem_ref):
    b = pl.program_id(0)
    nbatch = pl.num_programs(0)
    n = buf_ref.shape[1]
    c = buf_ref.shape[2]
    m = tcol_ref.shape[1]


    nchunk = sem_ref.shape[1]
    csz = n // nchunk

    def _start(bi, slot):
        for k in range(nchunk):
            sl = pl.ds(k * csz, csz)
            pltpu.make_async_copy(clas_hbm.at[bi, sl], buf_ref.at[slot, sl],
                                  sem_ref.at[slot, k]).start()

    @pl.when(b == 0)
    def _first():
        _start(0, 0)

    nxt = b + 1

    @pl.when(nxt < nbatch)
    def _prefetch():
        _start(nxt, nxt % 2)

    for k in range(nchunk):
        sl = pl.ds(k * csz, csz)
        pltpu.make_async_copy(clas_hbm.at[b, sl], buf_ref.at[b % 2, sl],
                              sem_ref.at[b % 2, k]).wait()

    rt_full = regs_ref[0]
    at_full = anc_ref[...]
    tc = tcol_ref[0]
    tr = trow_ref[0]

    tx0 = tc[:, 0:1]
    ty0 = tc[:, 1:2]
    tx1 = tc[:, 2:3]
    ty1 = tc[:, 3:4]
    area_b = (tx1 - tx0) * (ty1 - ty0)

    slot = b % 2
    ch = 512
    cls_sum = 0.0
    reg_sum = 0.0
    npos = 0.0
    for k0 in range(0, n, ch):
        w = min(ch, n - k0)
        cla_t = buf_ref[slot, k0:k0 + w, :].T
        at = at_full[:, k0:k0 + w]
        rt = rt_full[:, k0:k0 + w]

        ax0 = at[0:1, :]
        ay0 = at[1:2, :]
        ax1 = at[2:3, :]
        ay1 = at[3:4, :]

        iw = jnp.maximum(jnp.minimum(ax1, tx1) - jnp.maximum(ax0, tx0), 0.0)
        ih = jnp.maximum(jnp.minimum(ay1, ty1) - jnp.maximum(ay0, ty0), 0.0)
        inter = iw * ih
        area_a = (ax1 - ax0) * (ay1 - ay0)
        iou = inter / (area_a + area_b - inter)

        iou_max = jnp.max(iou, axis=0, keepdims=True)
        jio = jax.lax.broadcasted_iota(jnp.int32, (m, w), 0)
        amax = jnp.min(jnp.where(iou >= iou_max, jio, m), axis=0,
                       keepdims=True)
        sel = (jio == amax).astype(jnp.float32)

        pos = iou_max >= 0.5
        valid = jnp.logical_or(pos, iou_max < 0.4)

        matched = jax.lax.dot(tr, sel, precision=jax.lax.Precision.HIGHEST)
        mx0 = matched[0:1, :]
        my0 = matched[1:2, :]
        mx1 = matched[2:3, :]
        my1 = matched[3:4, :]
        cstar = matched[4:5, :].astype(jnp.int32)

        one_m = 1.0 - cla_t
        neg = (cla_t * cla_t) * jnp.log2(one_m)
        rowneg = (jnp.sum(neg, axis=0, keepdims=True)
                  * (-0.75 * 0.6931471805599453))
        cio = jax.lax.broadcasted_iota(jnp.int32, (c, w), 0)
        chosen = jnp.sum(jnp.where(cio == cstar, cla_t, 0.0), axis=0,
                         keepdims=True)
        och = 1.0 - chosen
        delta = (0.75 * chosen * chosen * jnp.log(och)
                 - 0.25 * och * och * jnp.log(chosen))
        cls_sum += jnp.sum(jnp.where(valid, rowneg, 0.0)
                           + jnp.where(pos, delta, 0.0))
        npos += jnp.sum(pos.astype(jnp.float32))

        aw = ax1 - ax0
        ah = ay1 - ay0
        gcx = ((mx0 + mx1) - (ax0 + ax1)) * 0.5 / (0.1 * aw)
        gcy = ((my0 + my1) - (ay0 + ay1)) * 0.5 / (0.1 * ah)
        gw = jnp.log((mx1 - mx0) / aw) * 5.0
        gh = jnp.log((my1 - my0) / ah) * 5.0
        rl = (_smooth_l1(jnp.abs(gcx - rt[0:1, :]))
              + _smooth_l1(jnp.abs(gcy - rt[1:2, :]))
              + _smooth_l1(jnp.abs(gw - rt[2:3, :]))
              + _smooth_l1(jnp.abs(gh - rt[3:4, :])))
        reg_sum += jnp.sum(jnp.where(pos, rl, 0.0))

    lane = jax.lax.broadcasted_iota(jnp.int32, (1, 1, 128), 2)
    out_ref[...] = (jnp.where(lane == 0, cls_sum, 0.0)
                    + jnp.where(lane == 1, reg_sum, 0.0)
                    + jnp.where(lane == 2, npos, 0.0))


def kernel(clas, regs, anchors, targets):
    b, n, c = clas.shape
    m = targets.shape[1]
    at = anchors[0].T
    rt = jnp.transpose(regs, (0, 2, 1))
    trow = jnp.transpose(targets, (0, 2, 1))

    out = pl.pallas_call(
        _body,
        grid=(b,),
        in_specs=[
            pl.BlockSpec(memory_space=pltpu.MemorySpace.HBM),
            pl.BlockSpec((1, 4, n), lambda i: (i, 0, 0)),
            pl.BlockSpec((4, n), lambda i: (0, 0)),
            pl.BlockSpec((1, m, 5), lambda i: (i, 0, 0)),
            pl.BlockSpec((1, 5, m), lambda i: (i, 0, 0)),
        ],
        out_specs=pl.BlockSpec((1, 1, 128), lambda i: (i, 0, 0)),
        out_shape=jax.ShapeDtypeStruct((b, 1, 128), jnp.float32),
        scratch_shapes=[
            pltpu.VMEM((2, n, c), jnp.float32),
            pltpu.SemaphoreType.DMA((2, 4)),
        ],
        compiler_params=pltpu.CompilerParams(
            dimension_semantics=("arbitrary",)),
    )(clas, rt, at, targets, trow)

    cls_sum = out[:, 0, 0]
    reg_sum = out[:, 0, 1]
    npos = out[:, 0, 2]
    cla_loss = jnp.mean(cls_sum / jnp.maximum(npos, 1.0)).reshape(1)
    rl_mean = reg_sum / jnp.maximum(npos * 4.0, 1.0)
    reg_loss = jnp.mean(jnp.where(npos > 0.0, rl_mean, 0.0)).reshape(1)
    return cla_loss, reg_loss


# speedup vs baseline: 1.1016x; 1.1016x over previous
import jax
import jax.numpy as jnp
from jax.experimental import pallas as pl
from jax.experimental.pallas import tpu as pltpu


def _smooth_l1(d):
    return jnp.where(d <= 1.0 / 9.0, 0.5 * 9.0 * d * d, d - 0.5 / 9.0)


def _body(clas_hbm, regs_ref, anc_ref, tcol_ref, trow_ref, out_ref,
          buf_ref, sem_ref):
    b = pl.program_id(0)
    nbatch = pl.num_programs(0)
    n = buf_ref.shape[1]
    c = buf_ref.shape[2]
    m = tcol_ref.shape[1]


    nchunk = sem_ref.shape[1]
    csz = n // nchunk

    def _start(bi, slot):
        for k in range(nchunk):
            sl = pl.ds(k * csz, csz)
            pltpu.make_async_copy(clas_hbm.at[bi, sl], buf_ref.at[slot, sl],
                                  sem_ref.at[slot, k]).start()

    @pl.when(b == 0)
    def _first():
        _start(0, 0)

    nxt = b + 1

    @pl.when(nxt < nbatch)
    def _prefetch():
        _start(nxt, nxt % 2)

    for k in range(nchunk):
        sl = pl.ds(k * csz, csz)
        pltpu.make_async_copy(clas_hbm.at[b, sl], buf_ref.at[b % 2, sl],
                              sem_ref.at[b % 2, k]).wait()

    rt_full = regs_ref[0]
    at_full = anc_ref[...]
    tc = tcol_ref[0]
    tr = trow_ref[0]

    tx0 = tc[:, 0:1]
    ty0 = tc[:, 1:2]
    tx1 = tc[:, 2:3]
    ty1 = tc[:, 3:4]
    area_b = (tx1 - tx0) * (ty1 - ty0)

    slot = b % 2
    ch = 512
    jio = jax.lax.broadcasted_iota(jnp.int32, (m, ch), 0)
    cio = jax.lax.broadcasted_iota(jnp.int32, (c, ch), 0)

    tr_hi = tr.astype(jnp.bfloat16)
    r1 = tr - tr_hi.astype(jnp.float32)
    tr_mid = r1.astype(jnp.bfloat16)
    tr_lo = (r1 - tr_mid.astype(jnp.float32)).astype(jnp.bfloat16)

    def _chunk(k0, w):
        cla_t = buf_ref[slot, k0:k0 + w, :].T
        at = at_full[:, k0:k0 + w]
        rt = rt_full[:, k0:k0 + w]

        ax0 = at[0:1, :]
        ay0 = at[1:2, :]
        ax1 = at[2:3, :]
        ay1 = at[3:4, :]

        iw = jnp.minimum(ax1, tx1) - jnp.maximum(ax0, tx0)
        ih = jnp.maximum(jnp.minimum(ay1, ty1) - jnp.maximum(ay0, ty0), 0.0)
        inter = iw * ih
        area_a = (ax1 - ax0) * (ay1 - ay0)
        iou = inter / (area_a + area_b - inter)

        iou_max = jnp.max(iou, axis=0, keepdims=True)
        amax = jnp.min(jnp.where(iou >= iou_max, jio[:, :w], m), axis=0,
                       keepdims=True)
        sel = (jio[:, :w] == amax).astype(jnp.bfloat16)

        pos = iou_max >= 0.5
        valid = jnp.logical_or(pos, iou_max < 0.4)

        matched = (jax.lax.dot(tr_hi, sel,
                               preferred_element_type=jnp.float32)
                   + jax.lax.dot(tr_mid, sel,
                                 preferred_element_type=jnp.float32)
                   + jax.lax.dot(tr_lo, sel,
                                 preferred_element_type=jnp.float32))
        mx0 = matched[0:1, :]
        my0 = matched[1:2, :]
        mx1 = matched[2:3, :]
        my1 = matched[3:4, :]
        cstar = matched[4:5, :].astype(jnp.int32)

        one_m = 1.0 - cla_t
        neg = (cla_t * cla_t) * jnp.log2(one_m)
        rowneg = jnp.sum(neg, axis=0, keepdims=True)
        chosen = jnp.sum(jnp.where(cio[:, :w] == cstar, cla_t, 0.0), axis=0,
                         keepdims=True)
        och = 1.0 - chosen
        delta = (0.75 * chosen * chosen * jnp.log(och)
                 - 0.25 * och * och * jnp.log(chosen))
        crow = (jnp.where(valid, rowneg, 0.0) * (-0.75 * 0.6931471805599453)
                + jnp.where(pos, delta, 0.0))
        nrow = pos.astype(jnp.float32)

        raw = 1.0 / (ax1 - ax0)
        rah = 1.0 / (ay1 - ay0)
        gcx = ((mx0 + mx1) - (ax0 + ax1)) * 5.0 * raw
        gcy = ((my0 + my1) - (ay0 + ay1)) * 5.0 * rah
        gw = jnp.log2((mx1 - mx0) * raw) * (5.0 * 0.6931471805599453)
        gh = jnp.log2((my1 - my0) * rah) * (5.0 * 0.6931471805599453)
        rl = (_smooth_l1(jnp.abs(gcx - rt[0:1, :]))
              + _smooth_l1(jnp.abs(gcy - rt[1:2, :]))
              + _smooth_l1(jnp.abs(gw - rt[2:3, :]))
              + _smooth_l1(jnp.abs(gh - rt[3:4, :])))
        rrow = jnp.where(pos, rl, 0.0)
        return crow, rrow, nrow

    nfull = (n // ch) * ch
    acc_c = jnp.zeros((1, ch), jnp.float32)
    acc_r = jnp.zeros((1, ch), jnp.float32)
    acc_n = jnp.zeros((1, ch), jnp.float32)
    for k0 in range(0, nfull, ch):
        crow, rrow, nrow = _chunk(k0, ch)
        acc_c += crow
        acc_r += rrow
        acc_n += nrow
    cls_sum = jnp.sum(acc_c)
    reg_sum = jnp.sum(acc_r)
    npos = jnp.sum(acc_n)
    if nfull < n:
        crow, rrow, nrow = _chunk(nfull, n - nfull)
        cls_sum += jnp.sum(crow)
        reg_sum += jnp.sum(rrow)
        npos += jnp.sum(nrow)

    lane = jax.lax.broadcasted_iota(jnp.int32, (1, 1, 128), 2)
    out_ref[...] = (jnp.where(lane == 0, cls_sum, 0.0)
                    + jnp.where(lane == 1, reg_sum, 0.0)
                    + jnp.where(lane == 2, npos, 0.0))


def kernel(clas, regs, anchors, targets):
    b, n, c = clas.shape
    m = targets.shape[1]
    at = anchors[0].T
    rt = jnp.transpose(regs, (0, 2, 1))
    trow = jnp.transpose(targets, (0, 2, 1))

    out = pl.pallas_call(
        _body,
        grid=(b,),
        in_specs=[
            pl.BlockSpec(memory_space=pltpu.MemorySpace.HBM),
            pl.BlockSpec((1, 4, n), lambda i: (i, 0, 0)),
            pl.BlockSpec((4, n), lambda i: (0, 0)),
            pl.BlockSpec((1, m, 5), lambda i: (i, 0, 0)),
            pl.BlockSpec((1, 5, m), lambda i: (i, 0, 0)),
        ],
        out_specs=pl.BlockSpec((1, 1, 128), lambda i: (i, 0, 0)),
        out_shape=jax.ShapeDtypeStruct((b, 1, 128), jnp.float32),
        scratch_shapes=[
            pltpu.VMEM((2, n, c), jnp.float32),
            pltpu.SemaphoreType.DMA((2, 4)),
        ],
        compiler_params=pltpu.CompilerParams(
            dimension_semantics=("arbitrary",)),
    )(clas, rt, at, targets, trow)

    cls_sum = out[:, 0, 0]
    reg_sum = out[:, 0, 1]
    npos = out[:, 0, 2]
    cla_loss = jnp.mean(cls_sum / jnp.maximum(npos, 1.0)).reshape(1)
    rl_mean = reg_sum / jnp.maximum(npos * 4.0, 1.0)
    reg_loss = jnp.mean(jnp.where(npos > 0.0, rl_mean, 0.0)).reshape(1)
    return cla_loss, reg_loss


# speedup vs baseline: 1.1392x; 1.0341x over previous
import jax
import jax.numpy as jnp
from jax.experimental import pallas as pl
from jax.experimental.pallas import tpu as pltpu


def _smooth_l1(d):
    return jnp.where(d <= 1.0 / 9.0, 0.5 * 9.0 * d * d, d - 0.5 / 9.0)


def _body(clas_hbm, regs_ref, anc_ref, tcol_ref, trow_ref, out_ref,
          buf_ref, sem_ref):
    b = pl.program_id(0)
    nbatch = pl.num_programs(0)
    n = buf_ref.shape[1]
    c = buf_ref.shape[2]
    m = tcol_ref.shape[1]


    nchunk = sem_ref.shape[1]
    csz = n // nchunk

    def _start(bi, slot):
        for k in range(nchunk):
            sl = pl.ds(k * csz, csz)
            pltpu.make_async_copy(clas_hbm.at[bi, sl], buf_ref.at[slot, sl],
                                  sem_ref.at[slot, k]).start()

    @pl.when(b == 0)
    def _first():
        _start(0, 0)

    nxt = b + 1

    @pl.when(nxt < nbatch)
    def _prefetch():
        _start(nxt, nxt % 2)

    for k in range(nchunk):
        sl = pl.ds(k * csz, csz)
        pltpu.make_async_copy(clas_hbm.at[b, sl], buf_ref.at[b % 2, sl],
                              sem_ref.at[b % 2, k]).wait()

    rt_full = regs_ref[0]
    at_full = anc_ref[...]
    tc = tcol_ref[0]
    tr = trow_ref[0]

    tx0 = tc[:, 0:1]
    ty0 = tc[:, 1:2]
    tx1 = tc[:, 2:3]
    ty1 = tc[:, 3:4]
    area_b = (tx1 - tx0) * (ty1 - ty0)

    slot = b % 2
    ch = 512
    jio = jax.lax.broadcasted_iota(jnp.int32, (m, ch), 0)
    cio = jax.lax.broadcasted_iota(jnp.int32, (c, ch), 0)

    tr_hi = tr.astype(jnp.bfloat16)
    r1 = tr - tr_hi.astype(jnp.float32)
    tr_mid = r1.astype(jnp.bfloat16)
    tr_lo = (r1 - tr_mid.astype(jnp.float32)).astype(jnp.bfloat16)

    def _chunk(k0, w):
        cla_t = buf_ref[slot, k0:k0 + w, :].T
        at = at_full[:, k0:k0 + w]
        rt = rt_full[:, k0:k0 + w]

        ax0 = at[0:1, :]
        ay0 = at[1:2, :]
        ax1 = at[2:3, :]
        ay1 = at[3:4, :]

        iw = jnp.minimum(ax1, tx1) - jnp.maximum(ax0, tx0)
        ih = jnp.maximum(jnp.minimum(ay1, ty1) - jnp.maximum(ay0, ty0), 0.0)
        inter = iw * ih
        area_a = (ax1 - ax0) * (ay1 - ay0)
        iou = inter / (area_a + area_b - inter)

        iou_max = jnp.max(iou, axis=0, keepdims=True)
        amax = jnp.min(jnp.where(iou >= iou_max, jio[:, :w], m), axis=0,
                       keepdims=True)
        sel = (jio[:, :w] == amax).astype(jnp.bfloat16)

        pos = iou_max >= 0.5
        valid = jnp.logical_or(pos, iou_max < 0.4)

        matched = (jax.lax.dot(tr_hi, sel,
                               preferred_element_type=jnp.float32)
                   + jax.lax.dot(tr_mid, sel,
                                 preferred_element_type=jnp.float32)
                   + jax.lax.dot(tr_lo, sel,
                                 preferred_element_type=jnp.float32))
        cstar = matched[4:5, :].astype(jnp.int32)

        one_m = 1.0 - cla_t
        neg = (cla_t * cla_t) * jnp.log2(one_m)
        rowneg = jnp.sum(neg, axis=0, keepdims=True)
        chosen = jnp.sum(jnp.where(cio[:, :w] == cstar, cla_t, 0.0), axis=0,
                         keepdims=True)
        och = 1.0 - chosen
        delta = (0.75 * chosen * chosen * jnp.log(och)
                 - 0.25 * och * och * jnp.log(chosen))
        crow = (jnp.where(valid, rowneg, 0.0) * (-0.75 * 0.6931471805599453)
                + jnp.where(pos, delta, 0.0))
        nrow = pos.astype(jnp.float32)

        m01 = matched[0:2, :] + matched[2:4, :]
        mwh = matched[2:4, :] - matched[0:2, :]
        a01 = at[0:2, :] + at[2:4, :]
        rwh = 1.0 / (at[2:4, :] - at[0:2, :])
        gc = (m01 - a01) * 5.0 * rwh
        gwh = jnp.log2(mwh * rwh) * (5.0 * 0.6931471805599453)
        rl2 = (_smooth_l1(jnp.abs(gc - rt[0:2, :]))
               + _smooth_l1(jnp.abs(gwh - rt[2:4, :])))
        rl = rl2[0:1, :] + rl2[1:2, :]
        rrow = jnp.where(pos, rl, 0.0)
        return crow, rrow, nrow

    nfull = (n // ch) * ch
    acc_c = jnp.zeros((1, ch), jnp.float32)
    acc_r = jnp.zeros((1, ch), jnp.float32)
    acc_n = jnp.zeros((1, ch), jnp.float32)
    for k0 in range(0, nfull, ch):
        crow, rrow, nrow = _chunk(k0, ch)
        acc_c += crow
        acc_r += rrow
        acc_n += nrow
    cls_sum = jnp.sum(acc_c)
    reg_sum = jnp.sum(acc_r)
    npos = jnp.sum(acc_n)
    if nfull < n:
        crow, rrow, nrow = _chunk(nfull, n - nfull)
        cls_sum += jnp.sum(crow)
        reg_sum += jnp.sum(rrow)
        npos += jnp.sum(nrow)

    lane = jax.lax.broadcasted_iota(jnp.int32, (1, 1, 128), 2)
    out_ref[...] = (jnp.where(lane == 0, cls_sum, 0.0)
                    + jnp.where(lane == 1, reg_sum, 0.0)
                    + jnp.where(lane == 2, npos, 0.0))


def kernel(clas, regs, anchors, targets):
    b, n, c = clas.shape
    m = targets.shape[1]
    at = anchors[0].T
    rt = jnp.transpose(regs, (0, 2, 1))
    trow = jnp.transpose(targets, (0, 2, 1))

    out = pl.pallas_call(
        _body,
        grid=(b,),
        in_specs=[
            pl.BlockSpec(memory_space=pltpu.MemorySpace.HBM),
            pl.BlockSpec((1, 4, n), lambda i: (i, 0, 0)),
            pl.BlockSpec((4, n), lambda i: (0, 0)),
            pl.BlockSpec((1, m, 5), lambda i: (i, 0, 0)),
            pl.BlockSpec((1, 5, m), lambda i: (i, 0, 0)),
        ],
        out_specs=pl.BlockSpec((1, 1, 128), lambda i: (i, 0, 0)),
        out_shape=jax.ShapeDtypeStruct((b, 1, 128), jnp.float32),
        scratch_shapes=[
            pltpu.VMEM((2, n, c), jnp.float32),
            pltpu.SemaphoreType.DMA((2, 4)),
        ],
        compiler_params=pltpu.CompilerParams(
            dimension_semantics=("arbitrary",)),
    )(clas, rt, at, targets, trow)

    cls_sum = out[:, 0, 0]
    reg_sum = out[:, 0, 1]
    npos = out[:, 0, 2]
    cla_loss = jnp.mean(cls_sum / jnp.maximum(npos, 1.0)).reshape(1)
    rl_mean = reg_sum / jnp.maximum(npos * 4.0, 1.0)
    reg_loss = jnp.mean(jnp.where(npos > 0.0, rl_mean, 0.0)).reshape(1)
    return cla_loss, reg_loss
